# Initial kernel scaffold; baseline (speedup 1.0000x reference)
import jax
import jax.numpy as jnp
from jax.experimental import pallas as pl


def kernel(q, pos_weight):
    raise NotImplementedError("write your pallas kernel here")


# speedup vs baseline: 2.0306x; 2.0306x over previous
import jax
import jax.numpy as jnp
from jax.experimental import pallas as pl


def _body(q_ref, pos_ref, o_ref):
    o_ref[...] = q_ref[...] + jnp.transpose(pos_ref[...])[None, :, :]


def kernel(q, pos_weight):
    B, D, T = q.shape
    Tt = 256
    return pl.pallas_call(
        _body,
        grid=(T // Tt,),
        in_specs=[
            pl.BlockSpec((B, D, Tt), lambda i: (0, 0, i)),
            pl.BlockSpec((Tt, D), lambda i: (i, 0)),
        ],
        out_specs=pl.BlockSpec((B, D, Tt), lambda i: (0, 0, i)),
        out_shape=jax.ShapeDtypeStruct((B, D, T), q.dtype),
    )(q, pos_weight)
